# baseline (speedup 1.0000x reference)
import functools

import jax
import jax.numpy as jnp
from jax.experimental import pallas as pl
from jax.experimental.pallas import tpu as pltpu

_HI = jax.lax.Precision.HIGHEST


def _dot(a, b):
    return jax.lax.dot_general(a, b, (((1,), (0,)), ((), ())),
                               preferred_element_type=jnp.float32,
                               precision=_HI)


def _spatial_kernel(*refs, inv2s2, cout, seq, ncat):
    q_ref, s_ref, f_ref = refs[:3]
    cat_refs = refs[3:3 + ncat]
    prefs = refs[3 + ncat:-1]
    o_ref = refs[-1]

    q = q_ref[...]
    s = s_ref[...]
    f = f_ref[...]
    qn = jnp.sum(q * q, axis=1, keepdims=True)
    sn = jnp.sum(s * s, axis=1, keepdims=True)
    a = jnp.concatenate([jnp.ones_like(qn), -2.0 * q, qn], axis=1)
    bm = jnp.concatenate([sn, s, jnp.ones_like(sn)], axis=1)
    d2 = jax.lax.dot_general(a, bm, (((1,), (1,)), ((), ())),
                             preferred_element_type=jnp.float32,
                             precision=_HI)
    w = jnp.exp(d2 * (-inv2s2))
    r = _dot(w, f)
    h = r[:, :cout] / (r[:, cout:cout + 1] + 1e-8)

    ci = 0
    k = 0
    for step in seq:
        if step == "cat":
            h = jnp.concatenate([h, cat_refs[ci][...]], axis=1)
            ci += 1
        else:
            has_bn, relu = step
            h = _dot(h, prefs[k][...]) + prefs[k + 1][...]
            k += 2
            if has_bn:
                h = h * prefs[k][...] + prefs[k + 1][...]
                k += 2
            if relu:
                h = jnp.maximum(h, 0.0)
    o_ref[...] = h


def _spatial_call(qpc, spc, fea, sigma, ops=()):
    Q = qpc.shape[0]
    S, C = fea.shape
    bq = min(Q, 512)
    f_aug = jnp.concatenate([fea, jnp.ones((S, 1), jnp.float32)], axis=1)
    inv2s2 = 1.0 / (2.0 * sigma * sigma)

    seq = []
    cats = []
    pargs = []
    cw = C
    for op in ops:
        if op[0] == "cat":
            arr = op[1]
            seq.append("cat")
            cats.append(arr)
            cw += arr.shape[1]
        else:
            _, p, relu = op
            has_bn = "gamma" in p
            seq.append((has_bn, relu))
            pargs.append(p["W"])
            pargs.append(p["b"].reshape(1, -1))
            if has_bn:
                pargs.append(p["gamma"].reshape(1, -1))
                pargs.append(p["beta"].reshape(1, -1))
            cw = p["W"].shape[1]

    cat_specs = [pl.BlockSpec((bq, arr.shape[1]), lambda i: (i, 0))
                 for arr in cats]
    parm_specs = [pl.BlockSpec(arr.shape, lambda i: (0,) * arr.ndim)
                  for arr in pargs]
    return pl.pallas_call(
        functools.partial(_spatial_kernel, inv2s2=inv2s2, cout=C,
                          seq=tuple(seq), ncat=len(cats)),
        grid=(Q // bq,),
        in_specs=[
            pl.BlockSpec((bq, 3), lambda i: (i, 0)),
            pl.BlockSpec((S, 3), lambda i: (0, 0)),
            pl.BlockSpec((S, C + 1), lambda i: (0, 0)),
        ] + cat_specs + parm_specs,
        out_specs=pl.BlockSpec((bq, cw), lambda i: (i, 0)),
        out_shape=jax.ShapeDtypeStruct((Q, cw), jnp.float32),
    )(qpc, spc, f_aug, *cats, *pargs)


def _fps2_kernel(pa_ref, pb_ref, sa_ref, sb_ref, oa_ref, ob_ref, *, n, lanes):
    xa = pa_ref[0, :, :]
    ya = pa_ref[1, :, :]
    za = pa_ref[2, :, :]
    xb = pb_ref[0, :, :]
    yb = pb_ref[1, :, :]
    zb = pb_ref[2, :, :]
    ii = jax.lax.broadcasted_iota(jnp.int32, (8, lanes), 0)
    jj = jax.lax.broadcasted_iota(jnp.int32, (8, lanes), 1)
    flat = ii * lanes + jj
    big = jnp.int32(2 ** 30)

    def load3(ref, idx):
        return ref[idx * 3], ref[idx * 3 + 1], ref[idx * 3 + 2]

    def store3(ref, i, x, y, z):
        ref[i * 3] = x
        ref[i * 3 + 1] = y
        ref[i * 3 + 2] = z

    def body(i, carry):
        da, la, db, lb = carry
        ax, ay, az = load3(sa_ref, la)
        bx, by, bz = load3(sb_ref, lb)
        store3(oa_ref, i, ax, ay, az)
        store3(ob_ref, i, bx, by, bz)
        da = jnp.minimum(da, (xa - ax) ** 2 + (ya - ay) ** 2 + (za - az) ** 2)
        db = jnp.minimum(db, (xb - bx) ** 2 + (yb - by) ** 2 + (zb - bz) ** 2)
        ma = jnp.max(da)
        mb = jnp.max(db)
        na = jnp.min(jnp.where(da == ma, flat, big))
        nb = jnp.min(jnp.where(db == mb, flat, big))
        return da, na, db, nb

    inf = jnp.full((8, lanes), jnp.inf, jnp.float32)
    _, la, _, lb = jax.lax.fori_loop(
        0, n - 1, body, (inf, jnp.int32(0), inf, jnp.int32(0)))
    ax, ay, az = load3(sa_ref, la)
    bx, by, bz = load3(sb_ref, lb)
    store3(oa_ref, n - 1, ax, ay, az)
    store3(ob_ref, n - 1, bx, by, bz)


def _fps2_call(pca, pcb, n):
    N = pca.shape[0]
    lanes = N // 8
    pra = pca.T.reshape(3, 8, lanes)
    prb = pcb.T.reshape(3, 8, lanes)
    oa, ob = pl.pallas_call(
        functools.partial(_fps2_kernel, n=n, lanes=lanes),
        in_specs=[
            pl.BlockSpec(memory_space=pltpu.VMEM),
            pl.BlockSpec(memory_space=pltpu.VMEM),
            pl.BlockSpec(memory_space=pltpu.SMEM),
            pl.BlockSpec(memory_space=pltpu.SMEM),
        ],
        out_specs=(pl.BlockSpec(memory_space=pltpu.SMEM),
                   pl.BlockSpec(memory_space=pltpu.SMEM)),
        out_shape=(jax.ShapeDtypeStruct((3 * n,), jnp.float32),
                   jax.ShapeDtypeStruct((3 * n,), jnp.float32)),
    )(pra, prb, pca.reshape(-1), pcb.reshape(-1))
    return oa.reshape(n, 3), ob.reshape(n, 3)


def _mlp_kernel(*refs, flags):
    x_ref = refs[0]
    o_ref = refs[-1]
    params = refs[1:-1]
    h = x_ref[...]
    k = 0
    for has_bn, relu in flags:
        h = _dot(h, params[k][...]) + params[k + 1][...]
        k += 2
        if has_bn:
            h = h * params[k][...] + params[k + 1][...]
            k += 2
        if relu:
            h = jnp.maximum(h, 0.0)
    o_ref[...] = h


def _mlp_call(x, layers, last_relu=True):
    N = x.shape[0]
    flags = []
    args = [x]
    for li, p in enumerate(layers):
        relu = last_relu or li < len(layers) - 1
        has_bn = "gamma" in p
        flags.append((has_bn, relu))
        args.append(p["W"])
        args.append(p["b"].reshape(1, -1))
        if has_bn:
            args.append(p["gamma"].reshape(1, -1))
            args.append(p["beta"].reshape(1, -1))
    cout = layers[-1]["W"].shape[1]
    return pl.pallas_call(
        functools.partial(_mlp_kernel, flags=tuple(flags)),
        out_shape=jax.ShapeDtypeStruct((N, cout), jnp.float32),
    )(*args)


_NPOINTS = 2048
_RADIUS = 0.02


def _lin(p, relu=True):
    return ("lin", p, relu)


def _forward(pc1, pc2, feature1, feature2, P):
    s0 = _RADIUS * 5.0
    s1 = _RADIUS * 4.0 * 5.0
    s2 = _RADIUS * 8.0 * 5.0
    s3 = _RADIUS * 16.0 * 5.0

    def enc0(pc, fea):
        x = _mlp_call(fea, [P["cc0_0"], P["cc0_1"]])
        return _spatial_call(pc, pc, x, s0,
                             ops=[_lin(P["cc0_2"]), _lin(P["cc0_3"])])

    l0_fea1 = enc0(pc1, feature1)
    l0_fea2 = enc0(pc2, feature2)
    l1_pc1, l1_pc2 = _fps2_call(pc1, pc2, _NPOINTS)

    def enc1(lpc, pc, fea):
        x = _spatial_call(lpc, pc, fea, s1,
                          ops=[("cat", lpc), _lin(P["cc1_0"]), _lin(P["cc1_1"])])
        return _spatial_call(lpc, lpc, x, s1,
                             ops=[_lin(P["cc1_2"]), _lin(P["cc1_3"])])

    l1_fea1 = enc1(l1_pc1, pc1, l0_fea1)
    l1_fea2 = enc1(l1_pc2, pc2, l0_fea2)
    l2_pc1, l2_pc2 = _fps2_call(l1_pc1, l1_pc2, _NPOINTS // 4)

    l2_fea1_ = _spatial_call(l2_pc1, l1_pc1, l1_fea1, s2,
                             ops=[("cat", l2_pc1), _lin(P["cc2_0"]), _lin(P["cc2_1"])])
    l2_fea2_ = _spatial_call(l2_pc2, l1_pc2, l1_fea2, s2,
                             ops=[("cat", l2_pc2), _lin(P["cc2_0"]), _lin(P["cc2_1"])])
    l2_fea1 = _spatial_call(l2_pc1, l2_pc2, l2_fea2_, s2,
                            ops=[_lin(P["cc2_pc2_1"]), ("cat", l2_fea1_),
                                 _lin(P["cc2_2"])])
    l2_fea2 = _mlp_call(l2_fea2_, [P["cc2_pc2_2"]])

    l3_pc1, l3_pc2 = _fps2_call(l2_pc1, l2_pc2, _NPOINTS // 16)
    l3_fea1_ = _spatial_call(l3_pc1, l2_pc1, l2_fea1, s3,
                             ops=[("cat", l3_pc1), _lin(P["cc3_0"]), _lin(P["cc3_1"])])
    l3_fea2_ = _spatial_call(l3_pc2, l2_pc2, l2_fea2, s3,
                             ops=[("cat", l3_pc2), _lin(P["cc3_0"]), _lin(P["cc3_1"])])
    l3_one = _spatial_call(l3_pc1, l3_pc2, l3_fea2_, s3,
                           ops=[("cat", l3_fea1_), _lin(P["cc3_2"])])
    w23 = P["cc2_3"]["W"]
    c_up = l3_one.shape[1]
    p23 = {"W": jnp.concatenate([w23[:c_up] + w23[c_up:2 * c_up],
                                 w23[2 * c_up:]], axis=0),
           "b": P["cc2_3"]["b"]}
    l2_fea1 = _spatial_call(l2_pc1, l3_pc1, l3_one, s2,
                            ops=[("cat", l2_fea1), _lin(p23),
                                 _lin(P["cc2_4"]), _lin(P["cc2_5"])])
    l1_fea1 = _spatial_call(l1_pc1, l2_pc1, l2_fea1, s1,
                            ops=[("cat", l1_fea1), _lin(P["cc1_4"]),
                                 _lin(P["cc1_5"]), _lin(P["cc1_6"])])
    flow = _spatial_call(pc1, l1_pc1, l1_fea1, s0,
                         ops=[("cat", l0_fea1), _lin(P["cc0_4"]),
                              _lin(P["cc0_5"]), _lin(P["cc0_6"]),
                              _lin(P["refine"], relu=False)])
    return flow


def kernel(pc1, pc2, feature1, feature2, params):
    flow = _forward(pc1[0], pc2[0], feature1[0], feature2[0], params)
    return (flow[None], None)


# speedup vs baseline: 1.1602x; 1.1602x over previous
import functools

import jax
import jax.numpy as jnp
from jax.experimental import pallas as pl
from jax.experimental.pallas import tpu as pltpu

_HI = jax.lax.Precision.HIGHEST


def _dot(a, b):
    return jax.lax.dot_general(a, b, (((1,), (0,)), ((), ())),
                               preferred_element_type=jnp.float32,
                               precision=_HI)


def _spatial_kernel(*refs, cout, seq, ncat):
    q_ref, st_ref, f_ref = refs[:3]
    cat_refs = refs[3:3 + ncat]
    prefs = refs[3 + ncat:-1]
    o_ref = refs[-1]

    f = f_ref[...]
    qx = q_ref[:, 0:1]
    qy = q_ref[:, 1:2]
    qz = q_ref[:, 2:3]
    sx = st_ref[0:1, :]
    sy = st_ref[1:2, :]
    sz = st_ref[2:3, :]
    dx = qx - sx
    dy = qy - sy
    dz = qz - sz
    w = jnp.exp(-(dx * dx + dy * dy + dz * dz))
    r = _dot(w, f)
    h = r[:, :cout] / (r[:, cout:cout + 1] + 1e-8)

    ci = 0
    k = 0
    for step in seq:
        if step == "cat":
            h = jnp.concatenate([h, cat_refs[ci][...]], axis=1)
            ci += 1
        else:
            has_bn, relu = step
            h = _dot(h, prefs[k][...]) + prefs[k + 1][...]
            k += 2
            if has_bn:
                h = h * prefs[k][...] + prefs[k + 1][...]
                k += 2
            if relu:
                h = jnp.maximum(h, 0.0)
    o_ref[...] = h


def _spatial_call(qpc, spc, fea, sigma, ops=()):
    Q = qpc.shape[0]
    S, C = fea.shape
    bq = min(Q, 512)
    f_aug = jnp.concatenate([fea, jnp.ones((S, 1), jnp.float32)], axis=1)
    scale = 1.0 / (sigma * jnp.sqrt(2.0).astype(jnp.float32))
    qs = qpc * scale
    st = (spc * scale).T

    seq = []
    cats = []
    pargs = []
    cw = C
    for op in ops:
        if op[0] == "cat":
            arr = op[1]
            seq.append("cat")
            cats.append(arr)
            cw += arr.shape[1]
        else:
            _, p, relu = op
            has_bn = "gamma" in p
            seq.append((has_bn, relu))
            pargs.append(p["W"])
            pargs.append(p["b"].reshape(1, -1))
            if has_bn:
                pargs.append(p["gamma"].reshape(1, -1))
                pargs.append(p["beta"].reshape(1, -1))
            cw = p["W"].shape[1]

    cat_specs = [pl.BlockSpec((bq, arr.shape[1]), lambda i: (i, 0))
                 for arr in cats]
    parm_specs = [pl.BlockSpec(arr.shape, lambda i: (0,) * arr.ndim)
                  for arr in pargs]
    return pl.pallas_call(
        functools.partial(_spatial_kernel, cout=C,
                          seq=tuple(seq), ncat=len(cats)),
        grid=(Q // bq,),
        in_specs=[
            pl.BlockSpec((bq, 3), lambda i: (i, 0)),
            pl.BlockSpec((3, S), lambda i: (0, 0)),
            pl.BlockSpec((S, C + 1), lambda i: (0, 0)),
        ] + cat_specs + parm_specs,
        out_specs=pl.BlockSpec((bq, cw), lambda i: (i, 0)),
        out_shape=jax.ShapeDtypeStruct((Q, cw), jnp.float32),
    )(qs, st, f_aug, *cats, *pargs)


def _fps2_kernel(pa_ref, pb_ref, sa_ref, sb_ref, oa_ref, ob_ref, *, n, lanes):
    xa = pa_ref[0, :, :]
    ya = pa_ref[1, :, :]
    za = pa_ref[2, :, :]
    xb = pb_ref[0, :, :]
    yb = pb_ref[1, :, :]
    zb = pb_ref[2, :, :]
    ii = jax.lax.broadcasted_iota(jnp.int32, (8, lanes), 0)
    jj = jax.lax.broadcasted_iota(jnp.int32, (8, lanes), 1)
    flat = ii * lanes + jj
    big = jnp.int32(2 ** 30)

    def load3(ref, idx):
        return ref[idx * 3], ref[idx * 3 + 1], ref[idx * 3 + 2]

    def store3(ref, i, x, y, z):
        ref[i * 3] = x
        ref[i * 3 + 1] = y
        ref[i * 3 + 2] = z

    def body(i, carry):
        da, la, db, lb = carry
        ax, ay, az = load3(sa_ref, la)
        bx, by, bz = load3(sb_ref, lb)
        store3(oa_ref, i, ax, ay, az)
        store3(ob_ref, i, bx, by, bz)
        da = jnp.minimum(da, (xa - ax) ** 2 + (ya - ay) ** 2 + (za - az) ** 2)
        db = jnp.minimum(db, (xb - bx) ** 2 + (yb - by) ** 2 + (zb - bz) ** 2)
        ma = jnp.max(da)
        mb = jnp.max(db)
        na = jnp.min(jnp.where(da == ma, flat, big))
        nb = jnp.min(jnp.where(db == mb, flat, big))
        return da, na, db, nb

    inf = jnp.full((8, lanes), jnp.inf, jnp.float32)
    _, la, _, lb = jax.lax.fori_loop(
        0, n - 1, body, (inf, jnp.int32(0), inf, jnp.int32(0)))
    ax, ay, az = load3(sa_ref, la)
    bx, by, bz = load3(sb_ref, lb)
    store3(oa_ref, n - 1, ax, ay, az)
    store3(ob_ref, n - 1, bx, by, bz)


def _fps2_call(pca, pcb, n):
    N = pca.shape[0]
    lanes = N // 8
    pra = pca.T.reshape(3, 8, lanes)
    prb = pcb.T.reshape(3, 8, lanes)
    oa, ob = pl.pallas_call(
        functools.partial(_fps2_kernel, n=n, lanes=lanes),
        in_specs=[
            pl.BlockSpec(memory_space=pltpu.VMEM),
            pl.BlockSpec(memory_space=pltpu.VMEM),
            pl.BlockSpec(memory_space=pltpu.SMEM),
            pl.BlockSpec(memory_space=pltpu.SMEM),
        ],
        out_specs=(pl.BlockSpec(memory_space=pltpu.SMEM),
                   pl.BlockSpec(memory_space=pltpu.SMEM)),
        out_shape=(jax.ShapeDtypeStruct((3 * n,), jnp.float32),
                   jax.ShapeDtypeStruct((3 * n,), jnp.float32)),
    )(pra, prb, pca.reshape(-1), pcb.reshape(-1))
    return oa.reshape(n, 3), ob.reshape(n, 3)


def _mlp_kernel(*refs, flags):
    x_ref = refs[0]
    o_ref = refs[-1]
    params = refs[1:-1]
    h = x_ref[...]
    k = 0
    for has_bn, relu in flags:
        h = _dot(h, params[k][...]) + params[k + 1][...]
        k += 2
        if has_bn:
            h = h * params[k][...] + params[k + 1][...]
            k += 2
        if relu:
            h = jnp.maximum(h, 0.0)
    o_ref[...] = h


def _mlp_call(x, layers, last_relu=True):
    N = x.shape[0]
    flags = []
    args = [x]
    for li, p in enumerate(layers):
        relu = last_relu or li < len(layers) - 1
        has_bn = "gamma" in p
        flags.append((has_bn, relu))
        args.append(p["W"])
        args.append(p["b"].reshape(1, -1))
        if has_bn:
            args.append(p["gamma"].reshape(1, -1))
            args.append(p["beta"].reshape(1, -1))
    cout = layers[-1]["W"].shape[1]
    return pl.pallas_call(
        functools.partial(_mlp_kernel, flags=tuple(flags)),
        out_shape=jax.ShapeDtypeStruct((N, cout), jnp.float32),
    )(*args)


_NPOINTS = 2048
_RADIUS = 0.02


def _lin(p, relu=True):
    return ("lin", p, relu)


def _forward(pc1, pc2, feature1, feature2, P):
    s0 = _RADIUS * 5.0
    s1 = _RADIUS * 4.0 * 5.0
    s2 = _RADIUS * 8.0 * 5.0
    s3 = _RADIUS * 16.0 * 5.0

    def enc0(pc, fea):
        x = _mlp_call(fea, [P["cc0_0"], P["cc0_1"]])
        return _spatial_call(pc, pc, x, s0,
                             ops=[_lin(P["cc0_2"]), _lin(P["cc0_3"])])

    l0_fea1 = enc0(pc1, feature1)
    l0_fea2 = enc0(pc2, feature2)
    l1_pc1, l1_pc2 = _fps2_call(pc1, pc2, _NPOINTS)

    def enc1(lpc, pc, fea):
        x = _spatial_call(lpc, pc, fea, s1,
                          ops=[("cat", lpc), _lin(P["cc1_0"]), _lin(P["cc1_1"])])
        return _spatial_call(lpc, lpc, x, s1,
                             ops=[_lin(P["cc1_2"]), _lin(P["cc1_3"])])

    l1_fea1 = enc1(l1_pc1, pc1, l0_fea1)
    l1_fea2 = enc1(l1_pc2, pc2, l0_fea2)
    l2_pc1, l2_pc2 = _fps2_call(l1_pc1, l1_pc2, _NPOINTS // 4)

    l2_fea1_ = _spatial_call(l2_pc1, l1_pc1, l1_fea1, s2,
                             ops=[("cat", l2_pc1), _lin(P["cc2_0"]), _lin(P["cc2_1"])])
    l2_fea2_ = _spatial_call(l2_pc2, l1_pc2, l1_fea2, s2,
                             ops=[("cat", l2_pc2), _lin(P["cc2_0"]), _lin(P["cc2_1"])])
    l2_fea1 = _spatial_call(l2_pc1, l2_pc2, l2_fea2_, s2,
                            ops=[_lin(P["cc2_pc2_1"]), ("cat", l2_fea1_),
                                 _lin(P["cc2_2"])])
    l2_fea2 = _mlp_call(l2_fea2_, [P["cc2_pc2_2"]])

    l3_pc1, l3_pc2 = _fps2_call(l2_pc1, l2_pc2, _NPOINTS // 16)
    l3_fea1_ = _spatial_call(l3_pc1, l2_pc1, l2_fea1, s3,
                             ops=[("cat", l3_pc1), _lin(P["cc3_0"]), _lin(P["cc3_1"])])
    l3_fea2_ = _spatial_call(l3_pc2, l2_pc2, l2_fea2, s3,
                             ops=[("cat", l3_pc2), _lin(P["cc3_0"]), _lin(P["cc3_1"])])
    l3_one = _spatial_call(l3_pc1, l3_pc2, l3_fea2_, s3,
                           ops=[("cat", l3_fea1_), _lin(P["cc3_2"])])
    w23 = P["cc2_3"]["W"]
    c_up = l3_one.shape[1]
    p23 = {"W": jnp.concatenate([w23[:c_up] + w23[c_up:2 * c_up],
                                 w23[2 * c_up:]], axis=0),
           "b": P["cc2_3"]["b"]}
    l2_fea1 = _spatial_call(l2_pc1, l3_pc1, l3_one, s2,
                            ops=[("cat", l2_fea1), _lin(p23),
                                 _lin(P["cc2_4"]), _lin(P["cc2_5"])])
    l1_fea1 = _spatial_call(l1_pc1, l2_pc1, l2_fea1, s1,
                            ops=[("cat", l1_fea1), _lin(P["cc1_4"]),
                                 _lin(P["cc1_5"]), _lin(P["cc1_6"])])
    flow = _spatial_call(pc1, l1_pc1, l1_fea1, s0,
                         ops=[("cat", l0_fea1), _lin(P["cc0_4"]),
                              _lin(P["cc0_5"]), _lin(P["cc0_6"]),
                              _lin(P["refine"], relu=False)])
    return flow


def kernel(pc1, pc2, feature1, feature2, params):
    flow = _forward(pc1[0], pc2[0], feature1[0], feature2[0], params)
    return (flow[None], None)


# speedup vs baseline: 1.5273x; 1.3164x over previous
import functools

import jax
import jax.numpy as jnp
from jax.experimental import pallas as pl
from jax.experimental.pallas import tpu as pltpu

_HI = jax.lax.Precision.HIGHEST


def _dot(a, b):
    return jax.lax.dot_general(a, b, (((1,), (0,)), ((), ())),
                               preferred_element_type=jnp.float32,
                               precision=_HI)


def _spatial_kernel(*refs, cout, seq, ncat):
    q_ref, st_ref, f_ref = refs[:3]
    cat_refs = refs[3:3 + ncat]
    prefs = refs[3 + ncat:-1]
    o_ref = refs[-1]

    f = f_ref[...]
    qx = q_ref[:, 0:1]
    qy = q_ref[:, 1:2]
    qz = q_ref[:, 2:3]
    sx = st_ref[0:1, :]
    sy = st_ref[1:2, :]
    sz = st_ref[2:3, :]
    dx = qx - sx
    dy = qy - sy
    dz = qz - sz
    w = jnp.exp(-(dx * dx + dy * dy + dz * dz))
    r = _dot(w, f)
    h = r[:, :cout] / (r[:, cout:cout + 1] + 1e-8)

    ci = 0
    k = 0
    for step in seq:
        if step == "cat":
            h = jnp.concatenate([h, cat_refs[ci][...]], axis=1)
            ci += 1
        else:
            has_bn, relu = step
            h = _dot(h, prefs[k][...]) + prefs[k + 1][...]
            k += 2
            if has_bn:
                h = h * prefs[k][...] + prefs[k + 1][...]
                k += 2
            if relu:
                h = jnp.maximum(h, 0.0)
    o_ref[...] = h


def _spatial_call(qpc, spc, fea, sigma, ops=()):
    Q = qpc.shape[0]
    S, C = fea.shape
    bq = min(Q, 512)
    f_aug = jnp.concatenate([fea, jnp.ones((S, 1), jnp.float32)], axis=1)
    scale = 1.0 / (sigma * jnp.sqrt(2.0).astype(jnp.float32))
    qs = qpc * scale
    st = (spc * scale).T

    seq = []
    cats = []
    pargs = []
    cw = C
    for op in ops:
        if op[0] == "cat":
            arr = op[1]
            seq.append("cat")
            cats.append(arr)
            cw += arr.shape[1]
        else:
            _, p, relu = op
            has_bn = "gamma" in p
            seq.append((has_bn, relu))
            pargs.append(p["W"])
            pargs.append(p["b"].reshape(1, -1))
            if has_bn:
                pargs.append(p["gamma"].reshape(1, -1))
                pargs.append(p["beta"].reshape(1, -1))
            cw = p["W"].shape[1]

    cat_specs = [pl.BlockSpec((bq, arr.shape[1]), lambda i: (i, 0))
                 for arr in cats]
    parm_specs = [pl.BlockSpec(arr.shape, lambda i: (0,) * arr.ndim)
                  for arr in pargs]
    return pl.pallas_call(
        functools.partial(_spatial_kernel, cout=C,
                          seq=tuple(seq), ncat=len(cats)),
        grid=(Q // bq,),
        in_specs=[
            pl.BlockSpec((bq, 3), lambda i: (i, 0)),
            pl.BlockSpec((3, S), lambda i: (0, 0)),
            pl.BlockSpec((S, C + 1), lambda i: (0, 0)),
        ] + cat_specs + parm_specs,
        out_specs=pl.BlockSpec((bq, cw), lambda i: (i, 0)),
        out_shape=jax.ShapeDtypeStruct((Q, cw), jnp.float32),
    )(qs, st, f_aug, *cats, *pargs)


def _fps2_kernel(pa_ref, pb_ref, sa_ref, sb_ref, oa_ref, ob_ref, *, n, lanes):
    xa = pa_ref[0, :, :]
    ya = pa_ref[1, :, :]
    za = pa_ref[2, :, :]
    xb = pb_ref[0, :, :]
    yb = pb_ref[1, :, :]
    zb = pb_ref[2, :, :]
    srow = jax.lax.broadcasted_iota(jnp.int32, (8, 1), 0) * lanes

    def argmax_flat(d):
        j = jnp.argmax(d, axis=1, keepdims=True).astype(jnp.int32)
        v = jnp.max(d, axis=1, keepdims=True)
        f = srow + j
        for k in (4, 2, 1):
            ov = pltpu.roll(v, k, 0)
            of = pltpu.roll(f, k, 0)
            take = (ov > v) | ((ov == v) & (of < f))
            v = jnp.where(take, ov, v)
            f = jnp.where(take, of, f)
        return f[0, 0]

    def load3(ref, idx):
        return ref[idx * 3], ref[idx * 3 + 1], ref[idx * 3 + 2]

    def store3(ref, i, x, y, z):
        ref[i * 3] = x
        ref[i * 3 + 1] = y
        ref[i * 3 + 2] = z

    def body(i, carry):
        da, la, db, lb = carry
        ax, ay, az = load3(sa_ref, la)
        bx, by, bz = load3(sb_ref, lb)
        store3(oa_ref, i, ax, ay, az)
        store3(ob_ref, i, bx, by, bz)
        da = jnp.minimum(da, (xa - ax) ** 2 + (ya - ay) ** 2 + (za - az) ** 2)
        db = jnp.minimum(db, (xb - bx) ** 2 + (yb - by) ** 2 + (zb - bz) ** 2)
        na = argmax_flat(da)
        nb = argmax_flat(db)
        return da, na, db, nb

    inf = jnp.full((8, lanes), jnp.inf, jnp.float32)
    _, la, _, lb = jax.lax.fori_loop(
        0, n - 1, body, (inf, jnp.int32(0), inf, jnp.int32(0)))
    ax, ay, az = load3(sa_ref, la)
    bx, by, bz = load3(sb_ref, lb)
    store3(oa_ref, n - 1, ax, ay, az)
    store3(ob_ref, n - 1, bx, by, bz)


def _fps2_call(pca, pcb, n):
    N = pca.shape[0]
    lanes = N // 8
    pra = pca.T.reshape(3, 8, lanes)
    prb = pcb.T.reshape(3, 8, lanes)
    oa, ob = pl.pallas_call(
        functools.partial(_fps2_kernel, n=n, lanes=lanes),
        in_specs=[
            pl.BlockSpec(memory_space=pltpu.VMEM),
            pl.BlockSpec(memory_space=pltpu.VMEM),
            pl.BlockSpec(memory_space=pltpu.SMEM),
            pl.BlockSpec(memory_space=pltpu.SMEM),
        ],
        out_specs=(pl.BlockSpec(memory_space=pltpu.SMEM),
                   pl.BlockSpec(memory_space=pltpu.SMEM)),
        out_shape=(jax.ShapeDtypeStruct((3 * n,), jnp.float32),
                   jax.ShapeDtypeStruct((3 * n,), jnp.float32)),
    )(pra, prb, pca.reshape(-1), pcb.reshape(-1))
    return oa.reshape(n, 3), ob.reshape(n, 3)


def _mlp_kernel(*refs, flags):
    x_ref = refs[0]
    o_ref = refs[-1]
    params = refs[1:-1]
    h = x_ref[...]
    k = 0
    for has_bn, relu in flags:
        h = _dot(h, params[k][...]) + params[k + 1][...]
        k += 2
        if has_bn:
            h = h * params[k][...] + params[k + 1][...]
            k += 2
        if relu:
            h = jnp.maximum(h, 0.0)
    o_ref[...] = h


def _mlp_call(x, layers, last_relu=True):
    N = x.shape[0]
    flags = []
    args = [x]
    for li, p in enumerate(layers):
        relu = last_relu or li < len(layers) - 1
        has_bn = "gamma" in p
        flags.append((has_bn, relu))
        args.append(p["W"])
        args.append(p["b"].reshape(1, -1))
        if has_bn:
            args.append(p["gamma"].reshape(1, -1))
            args.append(p["beta"].reshape(1, -1))
    cout = layers[-1]["W"].shape[1]
    return pl.pallas_call(
        functools.partial(_mlp_kernel, flags=tuple(flags)),
        out_shape=jax.ShapeDtypeStruct((N, cout), jnp.float32),
    )(*args)


_NPOINTS = 2048
_RADIUS = 0.02


def _lin(p, relu=True):
    return ("lin", p, relu)


def _forward(pc1, pc2, feature1, feature2, P):
    s0 = _RADIUS * 5.0
    s1 = _RADIUS * 4.0 * 5.0
    s2 = _RADIUS * 8.0 * 5.0
    s3 = _RADIUS * 16.0 * 5.0

    def enc0(pc, fea):
        x = _mlp_call(fea, [P["cc0_0"], P["cc0_1"]])
        return _spatial_call(pc, pc, x, s0,
                             ops=[_lin(P["cc0_2"]), _lin(P["cc0_3"])])

    l0_fea1 = enc0(pc1, feature1)
    l0_fea2 = enc0(pc2, feature2)
    l1_pc1, l1_pc2 = _fps2_call(pc1, pc2, _NPOINTS)

    def enc1(lpc, pc, fea):
        x = _spatial_call(lpc, pc, fea, s1,
                          ops=[("cat", lpc), _lin(P["cc1_0"]), _lin(P["cc1_1"])])
        return _spatial_call(lpc, lpc, x, s1,
                             ops=[_lin(P["cc1_2"]), _lin(P["cc1_3"])])

    l1_fea1 = enc1(l1_pc1, pc1, l0_fea1)
    l1_fea2 = enc1(l1_pc2, pc2, l0_fea2)
    l2_pc1, l2_pc2 = _fps2_call(l1_pc1, l1_pc2, _NPOINTS // 4)

    l2_fea1_ = _spatial_call(l2_pc1, l1_pc1, l1_fea1, s2,
                             ops=[("cat", l2_pc1), _lin(P["cc2_0"]), _lin(P["cc2_1"])])
    l2_fea2_ = _spatial_call(l2_pc2, l1_pc2, l1_fea2, s2,
                             ops=[("cat", l2_pc2), _lin(P["cc2_0"]), _lin(P["cc2_1"])])
    l2_fea1 = _spatial_call(l2_pc1, l2_pc2, l2_fea2_, s2,
                            ops=[_lin(P["cc2_pc2_1"]), ("cat", l2_fea1_),
                                 _lin(P["cc2_2"])])
    l2_fea2 = _mlp_call(l2_fea2_, [P["cc2_pc2_2"]])

    l3_pc1, l3_pc2 = _fps2_call(l2_pc1, l2_pc2, _NPOINTS // 16)
    l3_fea1_ = _spatial_call(l3_pc1, l2_pc1, l2_fea1, s3,
                             ops=[("cat", l3_pc1), _lin(P["cc3_0"]), _lin(P["cc3_1"])])
    l3_fea2_ = _spatial_call(l3_pc2, l2_pc2, l2_fea2, s3,
                             ops=[("cat", l3_pc2), _lin(P["cc3_0"]), _lin(P["cc3_1"])])
    l3_one = _spatial_call(l3_pc1, l3_pc2, l3_fea2_, s3,
                           ops=[("cat", l3_fea1_), _lin(P["cc3_2"])])
    w23 = P["cc2_3"]["W"]
    c_up = l3_one.shape[1]
    p23 = {"W": jnp.concatenate([w23[:c_up] + w23[c_up:2 * c_up],
                                 w23[2 * c_up:]], axis=0),
           "b": P["cc2_3"]["b"]}
    l2_fea1 = _spatial_call(l2_pc1, l3_pc1, l3_one, s2,
                            ops=[("cat", l2_fea1), _lin(p23),
                                 _lin(P["cc2_4"]), _lin(P["cc2_5"])])
    l1_fea1 = _spatial_call(l1_pc1, l2_pc1, l2_fea1, s1,
                            ops=[("cat", l1_fea1), _lin(P["cc1_4"]),
                                 _lin(P["cc1_5"]), _lin(P["cc1_6"])])
    flow = _spatial_call(pc1, l1_pc1, l1_fea1, s0,
                         ops=[("cat", l0_fea1), _lin(P["cc0_4"]),
                              _lin(P["cc0_5"]), _lin(P["cc0_6"]),
                              _lin(P["refine"], relu=False)])
    return flow


def kernel(pc1, pc2, feature1, feature2, params):
    flow = _forward(pc1[0], pc2[0], feature1[0], feature2[0], params)
    return (flow[None], None)


# speedup vs baseline: 1.7058x; 1.1169x over previous
import functools

import jax
import jax.numpy as jnp
from jax.experimental import pallas as pl
from jax.experimental.pallas import tpu as pltpu

_HI = jax.lax.Precision.HIGHEST


def _dot(a, b):
    return jax.lax.dot_general(a, b, (((1,), (0,)), ((), ())),
                               preferred_element_type=jnp.float32,
                               precision=_HI)


def _spatial_kernel(*refs, cout, seq, ncat):
    q_ref, st_ref, f_ref = refs[:3]
    cat_refs = refs[3:3 + ncat]
    prefs = refs[3 + ncat:-1]
    o_ref = refs[-1]

    f = f_ref[...]
    qx = q_ref[:, 0:1]
    qy = q_ref[:, 1:2]
    qz = q_ref[:, 2:3]
    sx = st_ref[0:1, :]
    sy = st_ref[1:2, :]
    sz = st_ref[2:3, :]
    dx = qx - sx
    dy = qy - sy
    dz = qz - sz
    w = jnp.exp(-(dx * dx + dy * dy + dz * dz))
    r = _dot(w, f)
    h = r[:, :cout] / (r[:, cout:cout + 1] + 1e-8)

    ci = 0
    k = 0
    for step in seq:
        if step == "cat":
            h = jnp.concatenate([h, cat_refs[ci][...]], axis=1)
            ci += 1
        else:
            has_bn, relu = step
            h = _dot(h, prefs[k][...]) + prefs[k + 1][...]
            k += 2
            if has_bn:
                h = h * prefs[k][...] + prefs[k + 1][...]
                k += 2
            if relu:
                h = jnp.maximum(h, 0.0)
    o_ref[...] = h


def _spatial_call(qpc, spc, fea, sigma, ops=()):
    Q = qpc.shape[0]
    S, C = fea.shape
    bq = min(Q, 512)
    f_aug = jnp.concatenate([fea, jnp.ones((S, 1), jnp.float32)], axis=1)
    scale = 1.0 / (sigma * jnp.sqrt(2.0).astype(jnp.float32))
    qs = qpc * scale
    st = (spc * scale).T

    seq = []
    cats = []
    pargs = []
    cw = C
    for op in ops:
        if op[0] == "cat":
            arr = op[1]
            seq.append("cat")
            cats.append(arr)
            cw += arr.shape[1]
        else:
            _, p, relu = op
            has_bn = "gamma" in p
            seq.append((has_bn, relu))
            pargs.append(p["W"])
            pargs.append(p["b"].reshape(1, -1))
            if has_bn:
                pargs.append(p["gamma"].reshape(1, -1))
                pargs.append(p["beta"].reshape(1, -1))
            cw = p["W"].shape[1]

    cat_specs = [pl.BlockSpec((bq, arr.shape[1]), lambda i: (i, 0))
                 for arr in cats]
    parm_specs = [pl.BlockSpec(arr.shape, lambda i: (0,) * arr.ndim)
                  for arr in pargs]
    return pl.pallas_call(
        functools.partial(_spatial_kernel, cout=C,
                          seq=tuple(seq), ncat=len(cats)),
        grid=(Q // bq,),
        in_specs=[
            pl.BlockSpec((bq, 3), lambda i: (i, 0)),
            pl.BlockSpec((3, S), lambda i: (0, 0)),
            pl.BlockSpec((S, C + 1), lambda i: (0, 0)),
        ] + cat_specs + parm_specs,
        out_specs=pl.BlockSpec((bq, cw), lambda i: (i, 0)),
        out_shape=jax.ShapeDtypeStruct((Q, cw), jnp.float32),
    )(qs, st, f_aug, *cats, *pargs)


def _fps2_kernel(pa_ref, pb_ref, sa_ref, sb_ref, oa_ref, ob_ref, *, n, rows):
    xa = pa_ref[0, :, :]
    ya = pa_ref[1, :, :]
    za = pa_ref[2, :, :]
    xb = pb_ref[0, :, :]
    yb = pb_ref[1, :, :]
    zb = pb_ref[2, :, :]
    srow = jax.lax.broadcasted_iota(jnp.int32, (rows, 1), 0) * 128

    def argmax_flat(d):
        j = jnp.argmax(d, axis=1, keepdims=True).astype(jnp.int32)
        v = jnp.max(d, axis=1, keepdims=True)
        f = srow + j
        k = rows // 2
        while k >= 1:
            ov = pltpu.roll(v, k, 0)
            of = pltpu.roll(f, k, 0)
            take = (ov > v) | ((ov == v) & (of < f))
            v = jnp.where(take, ov, v)
            f = jnp.where(take, of, f)
            k //= 2
        return f[0, 0]

    def load3(ref, idx):
        return ref[idx * 3], ref[idx * 3 + 1], ref[idx * 3 + 2]

    def store3(ref, i, x, y, z):
        ref[i * 3] = x
        ref[i * 3 + 1] = y
        ref[i * 3 + 2] = z

    def body(i, carry):
        da, la, db, lb = carry
        ax, ay, az = load3(sa_ref, la)
        bx, by, bz = load3(sb_ref, lb)
        store3(oa_ref, i, ax, ay, az)
        store3(ob_ref, i, bx, by, bz)
        da = jnp.minimum(da, (xa - ax) ** 2 + (ya - ay) ** 2 + (za - az) ** 2)
        db = jnp.minimum(db, (xb - bx) ** 2 + (yb - by) ** 2 + (zb - bz) ** 2)
        na = argmax_flat(da)
        nb = argmax_flat(db)
        return da, na, db, nb

    inf = jnp.full((rows, 128), jnp.inf, jnp.float32)
    _, la, _, lb = jax.lax.fori_loop(
        0, n - 1, body, (inf, jnp.int32(0), inf, jnp.int32(0)))
    ax, ay, az = load3(sa_ref, la)
    bx, by, bz = load3(sb_ref, lb)
    store3(oa_ref, n - 1, ax, ay, az)
    store3(ob_ref, n - 1, bx, by, bz)


def _fps2_call(pca, pcb, n):
    N = pca.shape[0]
    rows = N // 128
    pra = pca.T.reshape(3, rows, 128)
    prb = pcb.T.reshape(3, rows, 128)
    oa, ob = pl.pallas_call(
        functools.partial(_fps2_kernel, n=n, rows=rows),
        in_specs=[
            pl.BlockSpec(memory_space=pltpu.VMEM),
            pl.BlockSpec(memory_space=pltpu.VMEM),
            pl.BlockSpec(memory_space=pltpu.SMEM),
            pl.BlockSpec(memory_space=pltpu.SMEM),
        ],
        out_specs=(pl.BlockSpec(memory_space=pltpu.SMEM),
                   pl.BlockSpec(memory_space=pltpu.SMEM)),
        out_shape=(jax.ShapeDtypeStruct((3 * n,), jnp.float32),
                   jax.ShapeDtypeStruct((3 * n,), jnp.float32)),
    )(pra, prb, pca.reshape(-1), pcb.reshape(-1))
    return oa.reshape(n, 3), ob.reshape(n, 3)


def _mlp_kernel(*refs, flags):
    x_ref = refs[0]
    o_ref = refs[-1]
    params = refs[1:-1]
    h = x_ref[...]
    k = 0
    for has_bn, relu in flags:
        h = _dot(h, params[k][...]) + params[k + 1][...]
        k += 2
        if has_bn:
            h = h * params[k][...] + params[k + 1][...]
            k += 2
        if relu:
            h = jnp.maximum(h, 0.0)
    o_ref[...] = h


def _mlp_call(x, layers, last_relu=True):
    N = x.shape[0]
    flags = []
    args = [x]
    for li, p in enumerate(layers):
        relu = last_relu or li < len(layers) - 1
        has_bn = "gamma" in p
        flags.append((has_bn, relu))
        args.append(p["W"])
        args.append(p["b"].reshape(1, -1))
        if has_bn:
            args.append(p["gamma"].reshape(1, -1))
            args.append(p["beta"].reshape(1, -1))
    cout = layers[-1]["W"].shape[1]
    return pl.pallas_call(
        functools.partial(_mlp_kernel, flags=tuple(flags)),
        out_shape=jax.ShapeDtypeStruct((N, cout), jnp.float32),
    )(*args)


_NPOINTS = 2048
_RADIUS = 0.02


def _lin(p, relu=True):
    return ("lin", p, relu)


def _forward(pc1, pc2, feature1, feature2, P):
    s0 = _RADIUS * 5.0
    s1 = _RADIUS * 4.0 * 5.0
    s2 = _RADIUS * 8.0 * 5.0
    s3 = _RADIUS * 16.0 * 5.0

    def enc0(pc, fea):
        x = _mlp_call(fea, [P["cc0_0"], P["cc0_1"]])
        return _spatial_call(pc, pc, x, s0,
                             ops=[_lin(P["cc0_2"]), _lin(P["cc0_3"])])

    l0_fea1 = enc0(pc1, feature1)
    l0_fea2 = enc0(pc2, feature2)
    l1_pc1, l1_pc2 = _fps2_call(pc1, pc2, _NPOINTS)

    def enc1(lpc, pc, fea):
        x = _spatial_call(lpc, pc, fea, s1,
                          ops=[("cat", lpc), _lin(P["cc1_0"]), _lin(P["cc1_1"])])
        return _spatial_call(lpc, lpc, x, s1,
                             ops=[_lin(P["cc1_2"]), _lin(P["cc1_3"])])

    l1_fea1 = enc1(l1_pc1, pc1, l0_fea1)
    l1_fea2 = enc1(l1_pc2, pc2, l0_fea2)
    l2_pc1, l2_pc2 = _fps2_call(l1_pc1, l1_pc2, _NPOINTS // 4)

    l2_fea1_ = _spatial_call(l2_pc1, l1_pc1, l1_fea1, s2,
                             ops=[("cat", l2_pc1), _lin(P["cc2_0"]), _lin(P["cc2_1"])])
    l2_fea2_ = _spatial_call(l2_pc2, l1_pc2, l1_fea2, s2,
                             ops=[("cat", l2_pc2), _lin(P["cc2_0"]), _lin(P["cc2_1"])])
    l2_fea1 = _spatial_call(l2_pc1, l2_pc2, l2_fea2_, s2,
                            ops=[_lin(P["cc2_pc2_1"]), ("cat", l2_fea1_),
                                 _lin(P["cc2_2"])])
    l2_fea2 = _mlp_call(l2_fea2_, [P["cc2_pc2_2"]])

    l3_pc1, l3_pc2 = _fps2_call(l2_pc1, l2_pc2, _NPOINTS // 16)
    l3_fea1_ = _spatial_call(l3_pc1, l2_pc1, l2_fea1, s3,
                             ops=[("cat", l3_pc1), _lin(P["cc3_0"]), _lin(P["cc3_1"])])
    l3_fea2_ = _spatial_call(l3_pc2, l2_pc2, l2_fea2, s3,
                             ops=[("cat", l3_pc2), _lin(P["cc3_0"]), _lin(P["cc3_1"])])
    l3_one = _spatial_call(l3_pc1, l3_pc2, l3_fea2_, s3,
                           ops=[("cat", l3_fea1_), _lin(P["cc3_2"])])
    w23 = P["cc2_3"]["W"]
    c_up = l3_one.shape[1]
    p23 = {"W": jnp.concatenate([w23[:c_up] + w23[c_up:2 * c_up],
                                 w23[2 * c_up:]], axis=0),
           "b": P["cc2_3"]["b"]}
    l2_fea1 = _spatial_call(l2_pc1, l3_pc1, l3_one, s2,
                            ops=[("cat", l2_fea1), _lin(p23),
                                 _lin(P["cc2_4"]), _lin(P["cc2_5"])])
    l1_fea1 = _spatial_call(l1_pc1, l2_pc1, l2_fea1, s1,
                            ops=[("cat", l1_fea1), _lin(P["cc1_4"]),
                                 _lin(P["cc1_5"]), _lin(P["cc1_6"])])
    flow = _spatial_call(pc1, l1_pc1, l1_fea1, s0,
                         ops=[("cat", l0_fea1), _lin(P["cc0_4"]),
                              _lin(P["cc0_5"]), _lin(P["cc0_6"]),
                              _lin(P["refine"], relu=False)])
    return flow


def kernel(pc1, pc2, feature1, feature2, params):
    flow = _forward(pc1[0], pc2[0], feature1[0], feature2[0], params)
    return (flow[None], None)


# speedup vs baseline: 1.7688x; 1.0369x over previous
import functools

import jax
import jax.numpy as jnp
from jax.experimental import pallas as pl
from jax.experimental.pallas import tpu as pltpu

_HI = jax.lax.Precision.HIGHEST


def _dot(a, b):
    return jax.lax.dot_general(a, b, (((1,), (0,)), ((), ())),
                               preferred_element_type=jnp.float32,
                               precision=_HI)


def _spatial_kernel(*refs, cout, seq, ncat):
    q_ref, st_ref, f_ref = refs[:3]
    cat_refs = refs[3:3 + ncat]
    prefs = refs[3 + ncat:-1]
    o_ref = refs[-1]

    f = f_ref[...]
    qx = q_ref[:, 0:1]
    qy = q_ref[:, 1:2]
    qz = q_ref[:, 2:3]
    sx = st_ref[0:1, :]
    sy = st_ref[1:2, :]
    sz = st_ref[2:3, :]
    dx = qx - sx
    dy = qy - sy
    dz = qz - sz
    w = jnp.exp(-(dx * dx + dy * dy + dz * dz))
    r = _dot(w, f)
    h = r[:, :cout] / (r[:, cout:cout + 1] + 1e-8)

    ci = 0
    k = 0
    for step in seq:
        if step == "cat":
            h = jnp.concatenate([h, cat_refs[ci][...]], axis=1)
            ci += 1
        else:
            has_bn, relu = step
            h = _dot(h, prefs[k][...]) + prefs[k + 1][...]
            k += 2
            if has_bn:
                h = h * prefs[k][...] + prefs[k + 1][...]
                k += 2
            if relu:
                h = jnp.maximum(h, 0.0)
    o_ref[...] = h


def _spatial_call(qpc, spc, fea, sigma, ops=()):
    Q = qpc.shape[0]
    S, C = fea.shape
    bq = min(Q, 512)
    f_aug = jnp.concatenate([fea, jnp.ones((S, 1), jnp.float32)], axis=1)
    scale = 1.0 / (sigma * jnp.sqrt(2.0).astype(jnp.float32))
    qs = qpc * scale
    st = (spc * scale).T

    seq = []
    cats = []
    pargs = []
    cw = C
    for op in ops:
        if op[0] == "cat":
            arr = op[1]
            seq.append("cat")
            cats.append(arr)
            cw += arr.shape[1]
        else:
            _, p, relu = op
            has_bn = "gamma" in p
            seq.append((has_bn, relu))
            pargs.append(p["W"])
            pargs.append(p["b"].reshape(1, -1))
            if has_bn:
                pargs.append(p["gamma"].reshape(1, -1))
                pargs.append(p["beta"].reshape(1, -1))
            cw = p["W"].shape[1]

    cat_specs = [pl.BlockSpec((bq, arr.shape[1]), lambda i: (i, 0))
                 for arr in cats]
    parm_specs = [pl.BlockSpec(arr.shape, lambda i: (0,) * arr.ndim)
                  for arr in pargs]
    return pl.pallas_call(
        functools.partial(_spatial_kernel, cout=C,
                          seq=tuple(seq), ncat=len(cats)),
        grid=(Q // bq,),
        in_specs=[
            pl.BlockSpec((bq, 3), lambda i: (i, 0)),
            pl.BlockSpec((3, S), lambda i: (0, 0)),
            pl.BlockSpec((S, C + 1), lambda i: (0, 0)),
        ] + cat_specs + parm_specs,
        out_specs=pl.BlockSpec((bq, cw), lambda i: (i, 0)),
        out_shape=jax.ShapeDtypeStruct((Q, cw), jnp.float32),
    )(qs, st, f_aug, *cats, *pargs)


def _spatial_kernel_pair(*refs, cout, seq, ncat):
    q_ref, st_ref, f_ref = refs[:3]
    cat_refs = refs[3:3 + ncat]
    prefs = refs[3 + ncat:-1]
    o_ref = refs[-1]

    f = f_ref[0]
    q = q_ref[0]
    st = st_ref[0]
    qx = q[:, 0:1]
    qy = q[:, 1:2]
    qz = q[:, 2:3]
    sx = st[0:1, :]
    sy = st[1:2, :]
    sz = st[2:3, :]
    dx = qx - sx
    dy = qy - sy
    dz = qz - sz
    w = jnp.exp(-(dx * dx + dy * dy + dz * dz))
    r = _dot(w, f)
    h = r[:, :cout] / (r[:, cout:cout + 1] + 1e-8)

    ci = 0
    k = 0
    for step in seq:
        if step == "cat":
            h = jnp.concatenate([h, cat_refs[ci][0]], axis=1)
            ci += 1
        else:
            has_bn, relu = step
            h = _dot(h, prefs[k][...]) + prefs[k + 1][...]
            k += 2
            if has_bn:
                h = h * prefs[k][...] + prefs[k + 1][...]
                k += 2
            if relu:
                h = jnp.maximum(h, 0.0)
    o_ref[0] = h


def _spatial_pair_call(qpcs, spcs, feas, sigma, ops=()):
    _, Q, _ = qpcs.shape
    _, S, C = feas.shape
    bq = min(Q, 512)
    f_aug = jnp.concatenate([feas, jnp.ones((2, S, 1), jnp.float32)], axis=2)
    scale = 1.0 / (sigma * jnp.sqrt(2.0).astype(jnp.float32))
    qs = qpcs * scale
    st = (spcs * scale).transpose(0, 2, 1)

    seq = []
    cats = []
    pargs = []
    cw = C
    for op in ops:
        if op[0] == "cat":
            arr = op[1]
            seq.append("cat")
            cats.append(arr)
            cw += arr.shape[2]
        else:
            _, p, relu = op
            has_bn = "gamma" in p
            seq.append((has_bn, relu))
            pargs.append(p["W"])
            pargs.append(p["b"].reshape(1, -1))
            if has_bn:
                pargs.append(p["gamma"].reshape(1, -1))
                pargs.append(p["beta"].reshape(1, -1))
            cw = p["W"].shape[1]

    cat_specs = [pl.BlockSpec((1, bq, arr.shape[2]), lambda c, i: (c, i, 0))
                 for arr in cats]
    parm_specs = [pl.BlockSpec(arr.shape, lambda c, i: (0,) * arr.ndim)
                  for arr in pargs]
    return pl.pallas_call(
        functools.partial(_spatial_kernel_pair, cout=C,
                          seq=tuple(seq), ncat=len(cats)),
        grid=(2, Q // bq),
        in_specs=[
            pl.BlockSpec((1, bq, 3), lambda c, i: (c, i, 0)),
            pl.BlockSpec((1, 3, S), lambda c, i: (c, 0, 0)),
            pl.BlockSpec((1, S, C + 1), lambda c, i: (c, 0, 0)),
        ] + cat_specs + parm_specs,
        out_specs=pl.BlockSpec((1, bq, cw), lambda c, i: (c, i, 0)),
        out_shape=jax.ShapeDtypeStruct((2, Q, cw), jnp.float32),
    )(qs, st, f_aug, *cats, *pargs)


def _fps2_kernel(pa_ref, pb_ref, sa_ref, sb_ref, oa_ref, ob_ref, *, n, rows):
    xa = pa_ref[0, :, :]
    ya = pa_ref[1, :, :]
    za = pa_ref[2, :, :]
    xb = pb_ref[0, :, :]
    yb = pb_ref[1, :, :]
    zb = pb_ref[2, :, :]
    srow = jax.lax.broadcasted_iota(jnp.int32, (rows, 1), 0) * 128

    def argmax_flat(d):
        j = jnp.argmax(d, axis=1, keepdims=True).astype(jnp.int32)
        v = jnp.max(d, axis=1, keepdims=True)
        f = srow + j
        k = rows // 2
        while k >= 1:
            ov = pltpu.roll(v, k, 0)
            of = pltpu.roll(f, k, 0)
            take = (ov > v) | ((ov == v) & (of < f))
            v = jnp.where(take, ov, v)
            f = jnp.where(take, of, f)
            k //= 2
        return f[0, 0]

    def load3(ref, idx):
        return ref[idx * 3], ref[idx * 3 + 1], ref[idx * 3 + 2]

    def store3(ref, i, x, y, z):
        ref[i * 3] = x
        ref[i * 3 + 1] = y
        ref[i * 3 + 2] = z

    def body(i, carry):
        da, la, db, lb = carry
        ax, ay, az = load3(sa_ref, la)
        bx, by, bz = load3(sb_ref, lb)
        store3(oa_ref, i, ax, ay, az)
        store3(ob_ref, i, bx, by, bz)
        da = jnp.minimum(da, (xa - ax) ** 2 + (ya - ay) ** 2 + (za - az) ** 2)
        db = jnp.minimum(db, (xb - bx) ** 2 + (yb - by) ** 2 + (zb - bz) ** 2)
        na = argmax_flat(da)
        nb = argmax_flat(db)
        return da, na, db, nb

    inf = jnp.full((rows, 128), jnp.inf, jnp.float32)
    _, la, _, lb = jax.lax.fori_loop(
        0, n - 1, body, (inf, jnp.int32(0), inf, jnp.int32(0)))
    ax, ay, az = load3(sa_ref, la)
    bx, by, bz = load3(sb_ref, lb)
    store3(oa_ref, n - 1, ax, ay, az)
    store3(ob_ref, n - 1, bx, by, bz)


def _fps2_call(pca, pcb, n):
    N = pca.shape[0]
    rows = N // 128
    pra = pca.T.reshape(3, rows, 128)
    prb = pcb.T.reshape(3, rows, 128)
    oa, ob = pl.pallas_call(
        functools.partial(_fps2_kernel, n=n, rows=rows),
        in_specs=[
            pl.BlockSpec(memory_space=pltpu.VMEM),
            pl.BlockSpec(memory_space=pltpu.VMEM),
            pl.BlockSpec(memory_space=pltpu.SMEM),
            pl.BlockSpec(memory_space=pltpu.SMEM),
        ],
        out_specs=(pl.BlockSpec(memory_space=pltpu.SMEM),
                   pl.BlockSpec(memory_space=pltpu.SMEM)),
        out_shape=(jax.ShapeDtypeStruct((3 * n,), jnp.float32),
                   jax.ShapeDtypeStruct((3 * n,), jnp.float32)),
    )(pra, prb, pca.reshape(-1), pcb.reshape(-1))
    return oa.reshape(n, 3), ob.reshape(n, 3)


def _mlp_kernel(*refs, flags):
    x_ref = refs[0]
    o_ref = refs[-1]
    params = refs[1:-1]
    h = x_ref[...]
    k = 0
    for has_bn, relu in flags:
        h = _dot(h, params[k][...]) + params[k + 1][...]
        k += 2
        if has_bn:
            h = h * params[k][...] + params[k + 1][...]
            k += 2
        if relu:
            h = jnp.maximum(h, 0.0)
    o_ref[...] = h


def _mlp_call(x, layers, last_relu=True):
    N = x.shape[0]
    flags = []
    args = [x]
    for li, p in enumerate(layers):
        relu = last_relu or li < len(layers) - 1
        has_bn = "gamma" in p
        flags.append((has_bn, relu))
        args.append(p["W"])
        args.append(p["b"].reshape(1, -1))
        if has_bn:
            args.append(p["gamma"].reshape(1, -1))
            args.append(p["beta"].reshape(1, -1))
    cout = layers[-1]["W"].shape[1]
    return pl.pallas_call(
        functools.partial(_mlp_kernel, flags=tuple(flags)),
        out_shape=jax.ShapeDtypeStruct((N, cout), jnp.float32),
    )(*args)


_NPOINTS = 2048
_RADIUS = 0.02


def _lin(p, relu=True):
    return ("lin", p, relu)


def _forward(pc1, pc2, feature1, feature2, P):
    s0 = _RADIUS * 5.0
    s1 = _RADIUS * 4.0 * 5.0
    s2 = _RADIUS * 8.0 * 5.0
    s3 = _RADIUS * 16.0 * 5.0

    x12 = _mlp_call(jnp.concatenate([feature1, feature2], axis=0),
                    [P["cc0_0"], P["cc0_1"]])
    pcs = jnp.stack([pc1, pc2])
    l0_fea = _spatial_pair_call(pcs, pcs, x12.reshape(2, pc1.shape[0], -1), s0,
                                ops=[_lin(P["cc0_2"]), _lin(P["cc0_3"])])
    l0_fea1, l0_fea2 = l0_fea[0], l0_fea[1]
    l1_pc1, l1_pc2 = _fps2_call(pc1, pc2, _NPOINTS)

    lpcs = jnp.stack([l1_pc1, l1_pc2])
    x = _spatial_pair_call(lpcs, pcs, l0_fea, s1,
                           ops=[("cat", lpcs), _lin(P["cc1_0"]), _lin(P["cc1_1"])])
    l1_fea = _spatial_pair_call(lpcs, lpcs, x, s1,
                                ops=[_lin(P["cc1_2"]), _lin(P["cc1_3"])])
    l1_fea1, l1_fea2 = l1_fea[0], l1_fea[1]
    l2_pc1, l2_pc2 = _fps2_call(l1_pc1, l1_pc2, _NPOINTS // 4)

    l2pcs = jnp.stack([l2_pc1, l2_pc2])
    l2_fea_ = _spatial_pair_call(l2pcs, lpcs, l1_fea, s2,
                                 ops=[("cat", l2pcs), _lin(P["cc2_0"]),
                                      _lin(P["cc2_1"])])
    l2_fea1_, l2_fea2_ = l2_fea_[0], l2_fea_[1]
    l2_fea1 = _spatial_call(l2_pc1, l2_pc2, l2_fea2_, s2,
                            ops=[_lin(P["cc2_pc2_1"]), ("cat", l2_fea1_),
                                 _lin(P["cc2_2"])])
    l2_fea2 = _mlp_call(l2_fea2_, [P["cc2_pc2_2"]])

    l3_pc1, l3_pc2 = _fps2_call(l2_pc1, l2_pc2, _NPOINTS // 16)
    l3pcs = jnp.stack([l3_pc1, l3_pc2])
    l3_fea_ = _spatial_pair_call(l3pcs, l2pcs,
                                 jnp.stack([l2_fea1, l2_fea2]), s3,
                                 ops=[("cat", l3pcs), _lin(P["cc3_0"]),
                                      _lin(P["cc3_1"])])
    l3_fea1_, l3_fea2_ = l3_fea_[0], l3_fea_[1]
    l3_one = _spatial_call(l3_pc1, l3_pc2, l3_fea2_, s3,
                           ops=[("cat", l3_fea1_), _lin(P["cc3_2"])])
    w23 = P["cc2_3"]["W"]
    c_up = l3_one.shape[1]
    p23 = {"W": jnp.concatenate([w23[:c_up] + w23[c_up:2 * c_up],
                                 w23[2 * c_up:]], axis=0),
           "b": P["cc2_3"]["b"]}
    l2_fea1 = _spatial_call(l2_pc1, l3_pc1, l3_one, s2,
                            ops=[("cat", l2_fea1), _lin(p23),
                                 _lin(P["cc2_4"]), _lin(P["cc2_5"])])
    l1_fea1 = _spatial_call(l1_pc1, l2_pc1, l2_fea1, s1,
                            ops=[("cat", l1_fea1), _lin(P["cc1_4"]),
                                 _lin(P["cc1_5"]), _lin(P["cc1_6"])])
    flow = _spatial_call(pc1, l1_pc1, l1_fea1, s0,
                         ops=[("cat", l0_fea1), _lin(P["cc0_4"]),
                              _lin(P["cc0_5"]), _lin(P["cc0_6"]),
                              _lin(P["refine"], relu=False)])
    return flow


def kernel(pc1, pc2, feature1, feature2, params):
    flow = _forward(pc1[0], pc2[0], feature1[0], feature2[0], params)
    return (flow[None], None)


# speedup vs baseline: 2.3473x; 1.3270x over previous
import functools

import jax
import jax.numpy as jnp
from jax.experimental import pallas as pl
from jax.experimental.pallas import tpu as pltpu

_HI = jax.lax.Precision.DEFAULT


def _dot(a, b):
    return jax.lax.dot_general(a, b, (((1,), (0,)), ((), ())),
                               preferred_element_type=jnp.float32,
                               precision=_HI)


def _spatial_kernel(*refs, cout, seq, ncat):
    q_ref, st_ref, f_ref = refs[:3]
    cat_refs = refs[3:3 + ncat]
    prefs = refs[3 + ncat:-1]
    o_ref = refs[-1]

    f = f_ref[...]
    qx = q_ref[:, 0:1]
    qy = q_ref[:, 1:2]
    qz = q_ref[:, 2:3]
    sx = st_ref[0:1, :]
    sy = st_ref[1:2, :]
    sz = st_ref[2:3, :]
    dx = qx - sx
    dy = qy - sy
    dz = qz - sz
    w = jnp.exp(-(dx * dx + dy * dy + dz * dz))
    r = _dot(w, f)
    h = r[:, :cout] / (r[:, cout:cout + 1] + 1e-8)

    ci = 0
    k = 0
    for step in seq:
        if step == "cat":
            h = jnp.concatenate([h, cat_refs[ci][...]], axis=1)
            ci += 1
        else:
            has_bn, relu = step
            h = _dot(h, prefs[k][...]) + prefs[k + 1][...]
            k += 2
            if has_bn:
                h = h * prefs[k][...] + prefs[k + 1][...]
                k += 2
            if relu:
                h = jnp.maximum(h, 0.0)
    o_ref[...] = h


def _spatial_call(qpc, spc, fea, sigma, ops=()):
    Q = qpc.shape[0]
    S, C = fea.shape
    bq = min(Q, 512)
    f_aug = jnp.concatenate([fea, jnp.ones((S, 1), jnp.float32)], axis=1)
    scale = 1.0 / (sigma * jnp.sqrt(2.0).astype(jnp.float32))
    qs = qpc * scale
    st = (spc * scale).T

    seq = []
    cats = []
    pargs = []
    cw = C
    for op in ops:
        if op[0] == "cat":
            arr = op[1]
            seq.append("cat")
            cats.append(arr)
            cw += arr.shape[1]
        else:
            _, p, relu = op
            has_bn = "gamma" in p
            seq.append((has_bn, relu))
            pargs.append(p["W"])
            pargs.append(p["b"].reshape(1, -1))
            if has_bn:
                pargs.append(p["gamma"].reshape(1, -1))
                pargs.append(p["beta"].reshape(1, -1))
            cw = p["W"].shape[1]

    cat_specs = [pl.BlockSpec((bq, arr.shape[1]), lambda i: (i, 0))
                 for arr in cats]
    parm_specs = [pl.BlockSpec(arr.shape, lambda i: (0,) * arr.ndim)
                  for arr in pargs]
    return pl.pallas_call(
        functools.partial(_spatial_kernel, cout=C,
                          seq=tuple(seq), ncat=len(cats)),
        grid=(Q // bq,),
        in_specs=[
            pl.BlockSpec((bq, 3), lambda i: (i, 0)),
            pl.BlockSpec((3, S), lambda i: (0, 0)),
            pl.BlockSpec((S, C + 1), lambda i: (0, 0)),
        ] + cat_specs + parm_specs,
        out_specs=pl.BlockSpec((bq, cw), lambda i: (i, 0)),
        out_shape=jax.ShapeDtypeStruct((Q, cw), jnp.float32),
    )(qs, st, f_aug, *cats, *pargs)


def _spatial_kernel_pair(*refs, cout, seq, ncat):
    q_ref, st_ref, f_ref = refs[:3]
    cat_refs = refs[3:3 + ncat]
    prefs = refs[3 + ncat:-1]
    o_ref = refs[-1]

    f = f_ref[0]
    q = q_ref[0]
    st = st_ref[0]
    qx = q[:, 0:1]
    qy = q[:, 1:2]
    qz = q[:, 2:3]
    sx = st[0:1, :]
    sy = st[1:2, :]
    sz = st[2:3, :]
    dx = qx - sx
    dy = qy - sy
    dz = qz - sz
    w = jnp.exp(-(dx * dx + dy * dy + dz * dz))
    r = _dot(w, f)
    h = r[:, :cout] / (r[:, cout:cout + 1] + 1e-8)

    ci = 0
    k = 0
    for step in seq:
        if step == "cat":
            h = jnp.concatenate([h, cat_refs[ci][0]], axis=1)
            ci += 1
        else:
            has_bn, relu = step
            h = _dot(h, prefs[k][...]) + prefs[k + 1][...]
            k += 2
            if has_bn:
                h = h * prefs[k][...] + prefs[k + 1][...]
                k += 2
            if relu:
                h = jnp.maximum(h, 0.0)
    o_ref[0] = h


def _spatial_pair_call(qpcs, spcs, feas, sigma, ops=()):
    _, Q, _ = qpcs.shape
    _, S, C = feas.shape
    bq = min(Q, 512)
    f_aug = jnp.concatenate([feas, jnp.ones((2, S, 1), jnp.float32)], axis=2)
    scale = 1.0 / (sigma * jnp.sqrt(2.0).astype(jnp.float32))
    qs = qpcs * scale
    st = (spcs * scale).transpose(0, 2, 1)

    seq = []
    cats = []
    pargs = []
    cw = C
    for op in ops:
        if op[0] == "cat":
            arr = op[1]
            seq.append("cat")
            cats.append(arr)
            cw += arr.shape[2]
        else:
            _, p, relu = op
            has_bn = "gamma" in p
            seq.append((has_bn, relu))
            pargs.append(p["W"])
            pargs.append(p["b"].reshape(1, -1))
            if has_bn:
                pargs.append(p["gamma"].reshape(1, -1))
                pargs.append(p["beta"].reshape(1, -1))
            cw = p["W"].shape[1]

    cat_specs = [pl.BlockSpec((1, bq, arr.shape[2]), lambda c, i: (c, i, 0))
                 for arr in cats]
    parm_specs = [pl.BlockSpec(arr.shape, lambda c, i: (0,) * arr.ndim)
                  for arr in pargs]
    return pl.pallas_call(
        functools.partial(_spatial_kernel_pair, cout=C,
                          seq=tuple(seq), ncat=len(cats)),
        grid=(2, Q // bq),
        in_specs=[
            pl.BlockSpec((1, bq, 3), lambda c, i: (c, i, 0)),
            pl.BlockSpec((1, 3, S), lambda c, i: (c, 0, 0)),
            pl.BlockSpec((1, S, C + 1), lambda c, i: (c, 0, 0)),
        ] + cat_specs + parm_specs,
        out_specs=pl.BlockSpec((1, bq, cw), lambda c, i: (c, i, 0)),
        out_shape=jax.ShapeDtypeStruct((2, Q, cw), jnp.float32),
    )(qs, st, f_aug, *cats, *pargs)


def _fps2_kernel(pa_ref, pb_ref, sa_ref, sb_ref, oa_ref, ob_ref, *, n, rows):
    xa = pa_ref[0, :, :]
    ya = pa_ref[1, :, :]
    za = pa_ref[2, :, :]
    xb = pb_ref[0, :, :]
    yb = pb_ref[1, :, :]
    zb = pb_ref[2, :, :]
    srow = jax.lax.broadcasted_iota(jnp.int32, (rows, 1), 0) * 128

    def argmax_flat(d):
        j = jnp.argmax(d, axis=1, keepdims=True).astype(jnp.int32)
        v = jnp.max(d, axis=1, keepdims=True)
        f = srow + j
        k = rows // 2
        while k >= 1:
            ov = pltpu.roll(v, k, 0)
            of = pltpu.roll(f, k, 0)
            take = (ov > v) | ((ov == v) & (of < f))
            v = jnp.where(take, ov, v)
            f = jnp.where(take, of, f)
            k //= 2
        return f[0, 0]

    def load3(ref, idx):
        return ref[idx * 3], ref[idx * 3 + 1], ref[idx * 3 + 2]

    def store3(ref, i, x, y, z):
        ref[i * 3] = x
        ref[i * 3 + 1] = y
        ref[i * 3 + 2] = z

    def body(i, carry):
        da, la, db, lb = carry
        ax, ay, az = load3(sa_ref, la)
        bx, by, bz = load3(sb_ref, lb)
        store3(oa_ref, i, ax, ay, az)
        store3(ob_ref, i, bx, by, bz)
        da = jnp.minimum(da, (xa - ax) ** 2 + (ya - ay) ** 2 + (za - az) ** 2)
        db = jnp.minimum(db, (xb - bx) ** 2 + (yb - by) ** 2 + (zb - bz) ** 2)
        na = argmax_flat(da)
        nb = argmax_flat(db)
        return da, na, db, nb

    inf = jnp.full((rows, 128), jnp.inf, jnp.float32)
    _, la, _, lb = jax.lax.fori_loop(
        0, n - 1, body, (inf, jnp.int32(0), inf, jnp.int32(0)))
    ax, ay, az = load3(sa_ref, la)
    bx, by, bz = load3(sb_ref, lb)
    store3(oa_ref, n - 1, ax, ay, az)
    store3(ob_ref, n - 1, bx, by, bz)


def _fps2_call(pca, pcb, n):
    N = pca.shape[0]
    rows = N // 128
    pra = pca.T.reshape(3, rows, 128)
    prb = pcb.T.reshape(3, rows, 128)
    oa, ob = pl.pallas_call(
        functools.partial(_fps2_kernel, n=n, rows=rows),
        in_specs=[
            pl.BlockSpec(memory_space=pltpu.VMEM),
            pl.BlockSpec(memory_space=pltpu.VMEM),
            pl.BlockSpec(memory_space=pltpu.SMEM),
            pl.BlockSpec(memory_space=pltpu.SMEM),
        ],
        out_specs=(pl.BlockSpec(memory_space=pltpu.SMEM),
                   pl.BlockSpec(memory_space=pltpu.SMEM)),
        out_shape=(jax.ShapeDtypeStruct((3 * n,), jnp.float32),
                   jax.ShapeDtypeStruct((3 * n,), jnp.float32)),
    )(pra, prb, pca.reshape(-1), pcb.reshape(-1))
    return oa.reshape(n, 3), ob.reshape(n, 3)


def _mlp_kernel(*refs, flags):
    x_ref = refs[0]
    o_ref = refs[-1]
    params = refs[1:-1]
    h = x_ref[...]
    k = 0
    for has_bn, relu in flags:
        h = _dot(h, params[k][...]) + params[k + 1][...]
        k += 2
        if has_bn:
            h = h * params[k][...] + params[k + 1][...]
            k += 2
        if relu:
            h = jnp.maximum(h, 0.0)
    o_ref[...] = h


def _mlp_call(x, layers, last_relu=True):
    N = x.shape[0]
    flags = []
    args = [x]
    for li, p in enumerate(layers):
        relu = last_relu or li < len(layers) - 1
        has_bn = "gamma" in p
        flags.append((has_bn, relu))
        args.append(p["W"])
        args.append(p["b"].reshape(1, -1))
        if has_bn:
            args.append(p["gamma"].reshape(1, -1))
            args.append(p["beta"].reshape(1, -1))
    cout = layers[-1]["W"].shape[1]
    return pl.pallas_call(
        functools.partial(_mlp_kernel, flags=tuple(flags)),
        out_shape=jax.ShapeDtypeStruct((N, cout), jnp.float32),
    )(*args)


_NPOINTS = 2048
_RADIUS = 0.02


def _lin(p, relu=True):
    return ("lin", p, relu)


def _forward(pc1, pc2, feature1, feature2, P):
    s0 = _RADIUS * 5.0
    s1 = _RADIUS * 4.0 * 5.0
    s2 = _RADIUS * 8.0 * 5.0
    s3 = _RADIUS * 16.0 * 5.0

    x12 = _mlp_call(jnp.concatenate([feature1, feature2], axis=0),
                    [P["cc0_0"], P["cc0_1"]])
    pcs = jnp.stack([pc1, pc2])
    l0_fea = _spatial_pair_call(pcs, pcs, x12.reshape(2, pc1.shape[0], -1), s0,
                                ops=[_lin(P["cc0_2"]), _lin(P["cc0_3"])])
    l0_fea1, l0_fea2 = l0_fea[0], l0_fea[1]
    l1_pc1, l1_pc2 = _fps2_call(pc1, pc2, _NPOINTS)

    lpcs = jnp.stack([l1_pc1, l1_pc2])
    x = _spatial_pair_call(lpcs, pcs, l0_fea, s1,
                           ops=[("cat", lpcs), _lin(P["cc1_0"]), _lin(P["cc1_1"])])
    l1_fea = _spatial_pair_call(lpcs, lpcs, x, s1,
                                ops=[_lin(P["cc1_2"]), _lin(P["cc1_3"])])
    l1_fea1, l1_fea2 = l1_fea[0], l1_fea[1]
    l2_pc1, l2_pc2 = _fps2_call(l1_pc1, l1_pc2, _NPOINTS // 4)

    l2pcs = jnp.stack([l2_pc1, l2_pc2])
    l2_fea_ = _spatial_pair_call(l2pcs, lpcs, l1_fea, s2,
                                 ops=[("cat", l2pcs), _lin(P["cc2_0"]),
                                      _lin(P["cc2_1"])])
    l2_fea1_, l2_fea2_ = l2_fea_[0], l2_fea_[1]
    l2_fea1 = _spatial_call(l2_pc1, l2_pc2, l2_fea2_, s2,
                            ops=[_lin(P["cc2_pc2_1"]), ("cat", l2_fea1_),
                                 _lin(P["cc2_2"])])
    l2_fea2 = _mlp_call(l2_fea2_, [P["cc2_pc2_2"]])

    l3_pc1, l3_pc2 = _fps2_call(l2_pc1, l2_pc2, _NPOINTS // 16)
    l3pcs = jnp.stack([l3_pc1, l3_pc2])
    l3_fea_ = _spatial_pair_call(l3pcs, l2pcs,
                                 jnp.stack([l2_fea1, l2_fea2]), s3,
                                 ops=[("cat", l3pcs), _lin(P["cc3_0"]),
                                      _lin(P["cc3_1"])])
    l3_fea1_, l3_fea2_ = l3_fea_[0], l3_fea_[1]
    l3_one = _spatial_call(l3_pc1, l3_pc2, l3_fea2_, s3,
                           ops=[("cat", l3_fea1_), _lin(P["cc3_2"])])
    w23 = P["cc2_3"]["W"]
    c_up = l3_one.shape[1]
    p23 = {"W": jnp.concatenate([w23[:c_up] + w23[c_up:2 * c_up],
                                 w23[2 * c_up:]], axis=0),
           "b": P["cc2_3"]["b"]}
    l2_fea1 = _spatial_call(l2_pc1, l3_pc1, l3_one, s2,
                            ops=[("cat", l2_fea1), _lin(p23),
                                 _lin(P["cc2_4"]), _lin(P["cc2_5"])])
    l1_fea1 = _spatial_call(l1_pc1, l2_pc1, l2_fea1, s1,
                            ops=[("cat", l1_fea1), _lin(P["cc1_4"]),
                                 _lin(P["cc1_5"]), _lin(P["cc1_6"])])
    flow = _spatial_call(pc1, l1_pc1, l1_fea1, s0,
                         ops=[("cat", l0_fea1), _lin(P["cc0_4"]),
                              _lin(P["cc0_5"]), _lin(P["cc0_6"]),
                              _lin(P["refine"], relu=False)])
    return flow


def kernel(pc1, pc2, feature1, feature2, params):
    flow = _forward(pc1[0], pc2[0], feature1[0], feature2[0], params)
    return (flow[None], None)
